# Initial kernel scaffold; baseline (speedup 1.0000x reference)
import jax
import jax.numpy as jnp
from jax.experimental import pallas as pl


def kernel(x, edge_index, edge_type, idx, W0, root0, b0, W1, root1, b1, Wm, bm):
    raise NotImplementedError("write your pallas kernel here")


# speedup vs baseline: 2.8850x; 2.8850x over previous
import functools

import jax
import jax.numpy as jnp
from jax import lax
from jax.experimental import pallas as pl
from jax.experimental.pallas import tpu as pltpu
from jax.experimental.pallas import tpu_sc as plsc

_N = 10000
_E = 320000
_D = 128
_R = 2
_OUT = 5
_SEL = 2048

_N2 = _N // 2
_NLOC = 10240
_TRASH = 2 * _N2
_CH = 128
_K = 160
_G = 16
_EPW = _K * _CH
_EPAD = 16 * _EPW
_RPT = _NLOC // 16

_f32 = jnp.float32


def _mesh():
    return plsc.VectorSubcoreMesh(
        core_axis_name="c", subcore_axis_name="s", num_cores=2, num_subcores=16
    )


def _edge_pass_body(yp, gidx, sidx, zer, zer1, ones1,
                    agg_out, cnt_out,
                    gidx_v, sidx_v, rows_v, ones_v, agg_s, cnt_s):
    c = lax.axis_index("c")
    s = lax.axis_index("s")
    pltpu.sync_copy(zer, agg_s.at[pl.ds(s * _RPT, _RPT)])
    pltpu.sync_copy(zer1.at[pl.ds(s * _RPT, _RPT)], cnt_s.at[pl.ds(s * _RPT, _RPT)])
    pltpu.sync_copy(ones1, ones_v)
    plsc.subcore_barrier()

    def group(g, carry):
        pltpu.sync_copy(gidx.at[pl.ds(s * _K + g * _G, _G)], gidx_v)
        pltpu.sync_copy(sidx.at[c, pl.ds(s * _K + g * _G, _G)], sidx_v)

        def chunk(j, carry2):
            pltpu.sync_copy(yp.at[gidx_v.at[j]], rows_v)
            pltpu.sync_copy(rows_v, agg_s.at[sidx_v.at[j]], add=True)
            pltpu.sync_copy(ones_v, cnt_s.at[sidx_v.at[j]], add=True)
            return carry2

        lax.fori_loop(0, _G, chunk, 0)
        return carry

    lax.fori_loop(0, _K // _G, group, 0)
    plsc.subcore_barrier()
    pltpu.sync_copy(agg_s.at[pl.ds(s * _RPT, _RPT)],
                    agg_out.at[c, pl.ds(s * _RPT, _RPT)])
    pltpu.sync_copy(cnt_s.at[pl.ds(s * _RPT, _RPT)],
                    cnt_out.at[c, pl.ds(s * _RPT, _RPT)])


def _edge_pass(yp, gidx, sidx, zer, zer1, ones1):
    k = functools.partial(
        pl.kernel,
        out_type=[
            jax.ShapeDtypeStruct((2, _NLOC, _D), _f32),
            jax.ShapeDtypeStruct((2, _NLOC), _f32),
        ],
        mesh=_mesh(),
        scratch_types=[
            pltpu.VMEM((_G, _CH), jnp.int32),
            pltpu.VMEM((_G, _CH), jnp.int32),
            pltpu.VMEM((_CH, _D), _f32),
            pltpu.VMEM((_CH,), _f32),
            pltpu.VMEM_SHARED((_NLOC, _D), _f32),
            pltpu.VMEM_SHARED((_NLOC,), _f32),
        ],
    )(_edge_pass_body)
    return k(yp, gidx, sidx, zer, zer1, ones1)


def _gather_body(h, idx, out, idx_v, rows_v):
    wid = lax.axis_index("s") * 2 + lax.axis_index("c")
    base = wid * (_SEL // 32)
    pltpu.sync_copy(idx.at[pl.ds(base, _SEL // 32)], idx_v)
    pltpu.sync_copy(h.at[idx_v], rows_v)
    pltpu.sync_copy(rows_v, out.at[pl.ds(base, _SEL // 32)])


def _sel_gather(h, idx):
    k = functools.partial(
        pl.kernel,
        out_type=jax.ShapeDtypeStruct((_SEL, _D), _f32),
        mesh=_mesh(),
        scratch_types=[
            pltpu.VMEM((_SEL // 32,), jnp.int32),
            pltpu.VMEM((_SEL // 32, _D), _f32),
        ],
    )(_gather_body)
    return k(h, idx)


def _mm_body(h_ref, w0_ref, w1_ref, root_ref, b_ref, yp_ref, base_ref):
    hb = h_ref[...]
    yp_ref[0] = jnp.dot(hb, w0_ref[...], preferred_element_type=_f32)
    yp_ref[1] = jnp.dot(hb, w1_ref[...], preferred_element_type=_f32)
    base_ref[...] = jnp.dot(hb, root_ref[...], preferred_element_type=_f32) + b_ref[...]


_BN = 1000


def _mm(h, w0, w1, root, b):
    wspec = pl.BlockSpec((_D, _D), lambda i: (0, 0))
    return pl.pallas_call(
        _mm_body,
        grid=(_N // _BN,),
        in_specs=[
            pl.BlockSpec((_BN, _D), lambda i: (i, 0)),
            wspec, wspec, wspec,
            pl.BlockSpec((1, _D), lambda i: (0, 0)),
        ],
        out_specs=[
            pl.BlockSpec((2, _BN, _D), lambda i: (0, i, 0)),
            pl.BlockSpec((_BN, _D), lambda i: (i, 0)),
        ],
        out_shape=[
            jax.ShapeDtypeStruct((2, _N, _D), _f32),
            jax.ShapeDtypeStruct((_N, _D), _f32),
        ],
    )(h, w0, w1, root, b.reshape(1, _D))


def _combine_body(base_ref, sums_ref, cnt_ref, h_ref):
    sm = sums_ref[0]
    cn = cnt_ref[0]
    c0 = jnp.maximum(cn[:, 0:1], 1.0)
    c1 = jnp.maximum(cn[:, 1:2], 1.0)
    o = base_ref[...] + sm[:, :_D] / c0 + sm[:, _D:] / c1
    h_ref[...] = jnp.where(o >= 0.0, o, 0.01 * o)


_CB = 1000


def _combine(base, sums, cnt):
    return pl.pallas_call(
        _combine_body,
        grid=(2, _N2 // _CB),
        in_specs=[
            pl.BlockSpec((_CB, _D), lambda c, i: (c * (_N2 // _CB) + i, 0)),
            pl.BlockSpec((1, _CB, 2 * _D), lambda c, i: (c, i, 0)),
            pl.BlockSpec((1, _CB, 2), lambda c, i: (c, i, 0)),
        ],
        out_specs=pl.BlockSpec((_CB, _D), lambda c, i: (c * (_N2 // _CB) + i, 0)),
        out_shape=jax.ShapeDtypeStruct((_N, _D), _f32),
    )(base, sums, cnt)


def _head_body(hs_ref, wm_ref, bm_ref, o_ref):
    z = jnp.dot(hs_ref[...], wm_ref[...], preferred_element_type=_f32) + bm_ref[...]
    o_ref[...] = 1.0 / (1.0 + jnp.exp(-z))


def _head(hs, wm_pad, bm_pad):
    return pl.pallas_call(
        _head_body,
        grid=(1,),
        in_specs=[
            pl.BlockSpec((_SEL, _D), lambda i: (0, 0)),
            pl.BlockSpec((_D, _D), lambda i: (0, 0)),
            pl.BlockSpec((1, _D), lambda i: (0, 0)),
        ],
        out_specs=pl.BlockSpec((_SEL, _D), lambda i: (0, 0)),
        out_shape=jax.ShapeDtypeStruct((_SEL, _D), _f32),
    )(hs, wm_pad, bm_pad)


def kernel(x, edge_index, edge_type, idx, W0, root0, b0, W1, root1, b1, Wm, bm):
    src = edge_index[0]
    dst = edge_index[1]
    et = edge_type

    gidx = et * _N + src
    half = (dst >= _N2).astype(jnp.int32)
    lidx = (dst - half * _N2) * _R + et
    sidx0 = jnp.where(half == 0, lidx, _TRASH)
    sidx1 = jnp.where(half == 1, lidx, _TRASH)

    pad = _EPAD - _E
    gidx = jnp.concatenate([gidx, jnp.zeros((pad,), jnp.int32)]).reshape(16 * _K, _CH)
    sidx = jnp.stack([
        jnp.concatenate([sidx0, jnp.full((pad,), _TRASH, jnp.int32)]),
        jnp.concatenate([sidx1, jnp.full((pad,), _TRASH, jnp.int32)]),
    ]).reshape(2, 16 * _K, _CH)

    zer = jnp.zeros((_RPT, _D), _f32)
    zer1 = jnp.zeros((_NLOC,), _f32)
    ones1 = jnp.ones((_CH,), _f32)

    wm_pad = jnp.pad(Wm, ((0, 0), (0, _D - _OUT)))
    bm_pad = jnp.pad(bm, (0, _D - _OUT)).reshape(1, _D)

    h = x
    for (w, root, b) in ((W0, root0, b0), (W1, root1, b1)):
        yp, base = _mm(h, w[0], w[1], root, b)
        agg, cnt = _edge_pass(yp.reshape(2 * _N, _D), gidx, sidx, zer, zer1, ones1)
        sums = agg.reshape(2, _NLOC // 2, 2 * _D)
        cnts = cnt.reshape(2, _NLOC // 2, 2)
        h = _combine(base, sums, cnts)

    h_sel = _sel_gather(h, idx)
    out = _head(h_sel, wm_pad, bm_pad)[:, :_OUT]
    return (h_sel, out)


# speedup vs baseline: 2.9965x; 1.0386x over previous
import functools

import jax
import jax.numpy as jnp
from jax import lax
from jax.experimental import pallas as pl
from jax.experimental.pallas import tpu as pltpu
from jax.experimental.pallas import tpu_sc as plsc

_N = 10000
_E = 320000
_D = 128
_R = 2
_OUT = 5
_SEL = 2048

_N2 = _N // 2
_NLOC = 10240
_TRASH = 2 * _N2
_CH = 128
_K = 160
_G = 16
_EPW = _K * _CH
_EPAD = 16 * _EPW
_RPT = _NLOC // 16

_f32 = jnp.float32


def _mesh():
    return plsc.VectorSubcoreMesh(
        core_axis_name="c", subcore_axis_name="s", num_cores=2, num_subcores=16
    )


_SG = 32
_NSG = _K // _SG


def _edge_pass_body(yp, gidx, sidx, zer, zer1, ones1,
                    agg_out, cnt_out,
                    gidx_v, sidx_v, rows_a, rows_b, ones_v,
                    sem_g0, sem_g1, sem_s0, sem_s1, sem_c,
                    agg_s, cnt_s):
    c = lax.axis_index("c")
    s = lax.axis_index("s")
    pltpu.sync_copy(zer, agg_s.at[pl.ds(s * _RPT, _RPT)])
    pltpu.sync_copy(zer1.at[pl.ds(s * _RPT, _RPT)], cnt_s.at[pl.ds(s * _RPT, _RPT)])
    pltpu.sync_copy(ones1, ones_v)
    plsc.subcore_barrier()

    rows = (rows_a, rows_b)
    sem_g = (sem_g0, sem_g1)
    sem_s = (sem_s0, sem_s1)

    def super_group(g, carry):
        base = s * _K + g * _SG
        pltpu.sync_copy(gidx.at[pl.ds(base, _SG)], gidx_v)
        pltpu.sync_copy(sidx.at[c, pl.ds(base, _SG)], sidx_v)

        gath = [None, None]
        scat = [None, None]
        cnts = []
        gath[0] = pltpu.async_copy(yp.at[gidx_v.at[0]], rows[0], sem_g[0])
        for j in range(1, _SG + 1):
            b = j & 1
            if j < _SG:
                if scat[b] is not None:
                    scat[b].wait()
                gath[b] = pltpu.async_copy(yp.at[gidx_v.at[j]], rows[b], sem_g[b])
            p = 1 - b
            gath[p].wait()
            scat[p] = pltpu.async_copy(
                rows[p], agg_s.at[sidx_v.at[j - 1]], sem_s[p], add=True)
            cnts.append(pltpu.async_copy(
                ones_v, cnt_s.at[sidx_v.at[j - 1]], sem_c, add=True))
        scat[0].wait()
        scat[1].wait()
        for d in cnts:
            d.wait()
        return carry

    lax.fori_loop(0, _NSG, super_group, 0)
    plsc.subcore_barrier()
    pltpu.sync_copy(agg_s.at[pl.ds(s * _RPT, _RPT)],
                    agg_out.at[c, pl.ds(s * _RPT, _RPT)])
    pltpu.sync_copy(cnt_s.at[pl.ds(s * _RPT, _RPT)],
                    cnt_out.at[c, pl.ds(s * _RPT, _RPT)])


def _edge_pass(yp, gidx, sidx, zer, zer1, ones1):
    k = functools.partial(
        pl.kernel,
        out_type=[
            jax.ShapeDtypeStruct((2, _NLOC, _D), _f32),
            jax.ShapeDtypeStruct((2, _NLOC), _f32),
        ],
        mesh=_mesh(),
        scratch_types=[
            pltpu.VMEM((_SG, _CH), jnp.int32),
            pltpu.VMEM((_SG, _CH), jnp.int32),
            pltpu.VMEM((_CH, _D), _f32),
            pltpu.VMEM((_CH, _D), _f32),
            pltpu.VMEM((_CH,), _f32),
            pltpu.SemaphoreType.DMA,
            pltpu.SemaphoreType.DMA,
            pltpu.SemaphoreType.DMA,
            pltpu.SemaphoreType.DMA,
            pltpu.SemaphoreType.DMA,
            pltpu.VMEM_SHARED((_NLOC, _D), _f32),
            pltpu.VMEM_SHARED((_NLOC,), _f32),
        ],
    )(_edge_pass_body)
    return k(yp, gidx, sidx, zer, zer1, ones1)


def _gather_body(h, idx, out, idx_v, rows_v):
    wid = lax.axis_index("s") * 2 + lax.axis_index("c")
    base = wid * (_SEL // 32)
    pltpu.sync_copy(idx.at[pl.ds(base, _SEL // 32)], idx_v)
    pltpu.sync_copy(h.at[idx_v], rows_v)
    pltpu.sync_copy(rows_v, out.at[pl.ds(base, _SEL // 32)])


def _sel_gather(h, idx):
    k = functools.partial(
        pl.kernel,
        out_type=jax.ShapeDtypeStruct((_SEL, _D), _f32),
        mesh=_mesh(),
        scratch_types=[
            pltpu.VMEM((_SEL // 32,), jnp.int32),
            pltpu.VMEM((_SEL // 32, _D), _f32),
        ],
    )(_gather_body)
    return k(h, idx)


def _mm_body(h_ref, w0_ref, w1_ref, root_ref, b_ref, yp_ref, base_ref):
    hb = h_ref[...]
    yp_ref[0] = jnp.dot(hb, w0_ref[...], preferred_element_type=_f32)
    yp_ref[1] = jnp.dot(hb, w1_ref[...], preferred_element_type=_f32)
    base_ref[...] = jnp.dot(hb, root_ref[...], preferred_element_type=_f32) + b_ref[...]


_BN = 1000


def _mm(h, w0, w1, root, b):
    wspec = pl.BlockSpec((_D, _D), lambda i: (0, 0))
    return pl.pallas_call(
        _mm_body,
        grid=(_N // _BN,),
        in_specs=[
            pl.BlockSpec((_BN, _D), lambda i: (i, 0)),
            wspec, wspec, wspec,
            pl.BlockSpec((1, _D), lambda i: (0, 0)),
        ],
        out_specs=[
            pl.BlockSpec((2, _BN, _D), lambda i: (0, i, 0)),
            pl.BlockSpec((_BN, _D), lambda i: (i, 0)),
        ],
        out_shape=[
            jax.ShapeDtypeStruct((2, _N, _D), _f32),
            jax.ShapeDtypeStruct((_N, _D), _f32),
        ],
    )(h, w0, w1, root, b.reshape(1, _D))


def _combine_body(base_ref, sums_ref, cnt_ref, h_ref):
    sm = sums_ref[0]
    cn = cnt_ref[0]
    c0 = jnp.maximum(cn[:, 0:1], 1.0)
    c1 = jnp.maximum(cn[:, 1:2], 1.0)
    o = base_ref[...] + sm[:, :_D] / c0 + sm[:, _D:] / c1
    h_ref[...] = jnp.where(o >= 0.0, o, 0.01 * o)


_CB = 1000


def _combine(base, sums, cnt):
    return pl.pallas_call(
        _combine_body,
        grid=(2, _N2 // _CB),
        in_specs=[
            pl.BlockSpec((_CB, _D), lambda c, i: (c * (_N2 // _CB) + i, 0)),
            pl.BlockSpec((1, _CB, 2 * _D), lambda c, i: (c, i, 0)),
            pl.BlockSpec((1, _CB, 2), lambda c, i: (c, i, 0)),
        ],
        out_specs=pl.BlockSpec((_CB, _D), lambda c, i: (c * (_N2 // _CB) + i, 0)),
        out_shape=jax.ShapeDtypeStruct((_N, _D), _f32),
    )(base, sums, cnt)


def _head_body(hs_ref, wm_ref, bm_ref, o_ref):
    z = jnp.dot(hs_ref[...], wm_ref[...], preferred_element_type=_f32) + bm_ref[...]
    o_ref[...] = 1.0 / (1.0 + jnp.exp(-z))


def _head(hs, wm_pad, bm_pad):
    return pl.pallas_call(
        _head_body,
        grid=(1,),
        in_specs=[
            pl.BlockSpec((_SEL, _D), lambda i: (0, 0)),
            pl.BlockSpec((_D, _D), lambda i: (0, 0)),
            pl.BlockSpec((1, _D), lambda i: (0, 0)),
        ],
        out_specs=pl.BlockSpec((_SEL, _D), lambda i: (0, 0)),
        out_shape=jax.ShapeDtypeStruct((_SEL, _D), _f32),
    )(hs, wm_pad, bm_pad)


def kernel(x, edge_index, edge_type, idx, W0, root0, b0, W1, root1, b1, Wm, bm):
    src = edge_index[0]
    dst = edge_index[1]
    et = edge_type

    gidx = et * _N + src
    half = (dst >= _N2).astype(jnp.int32)
    lidx = (dst - half * _N2) * _R + et
    sidx0 = jnp.where(half == 0, lidx, _TRASH)
    sidx1 = jnp.where(half == 1, lidx, _TRASH)

    pad = _EPAD - _E
    gidx = jnp.concatenate([gidx, jnp.zeros((pad,), jnp.int32)]).reshape(16 * _K, _CH)
    sidx = jnp.stack([
        jnp.concatenate([sidx0, jnp.full((pad,), _TRASH, jnp.int32)]),
        jnp.concatenate([sidx1, jnp.full((pad,), _TRASH, jnp.int32)]),
    ]).reshape(2, 16 * _K, _CH)

    zer = jnp.zeros((_RPT, _D), _f32)
    zer1 = jnp.zeros((_NLOC,), _f32)
    ones1 = jnp.ones((_CH,), _f32)

    wm_pad = jnp.pad(Wm, ((0, 0), (0, _D - _OUT)))
    bm_pad = jnp.pad(bm, (0, _D - _OUT)).reshape(1, _D)

    h = x
    for (w, root, b) in ((W0, root0, b0), (W1, root1, b1)):
        yp, base = _mm(h, w[0], w[1], root, b)
        agg, cnt = _edge_pass(yp.reshape(2 * _N, _D), gidx, sidx, zer, zer1, ones1)
        sums = agg.reshape(2, _NLOC // 2, 2 * _D)
        cnts = cnt.reshape(2, _NLOC // 2, 2)
        h = _combine(base, sums, cnts)

    h_sel = _sel_gather(h, idx)
    out = _head(h_sel, wm_pad, bm_pad)[:, :_OUT]
    return (h_sel, out)


# speedup vs baseline: 3.2278x; 1.0772x over previous
import functools

import jax
import jax.numpy as jnp
from jax import lax
from jax.experimental import pallas as pl
from jax.experimental.pallas import tpu as pltpu
from jax.experimental.pallas import tpu_sc as plsc

_N = 10000
_E = 320000
_D = 128
_R = 2
_OUT = 5
_SEL = 2048

_N2 = _N // 2
_NLOC = 10240
_TRASH = 2 * _N2
_CH = 128
_K = 160
_G = 16
_EPW = _K * _CH
_EPAD = 16 * _EPW
_RPT = _NLOC // 16

_f32 = jnp.float32


def _mesh():
    return plsc.VectorSubcoreMesh(
        core_axis_name="c", subcore_axis_name="s", num_cores=2, num_subcores=16
    )


_SG = 32
_NSG = _K // _SG


def _make_edge_pass(with_counts):
    def body(yp, gidx, sidx, zer, zer1, ones1, *rest):
        if with_counts:
            (agg_out, cnt_out,
             gidx_v, sidx_v, rows_a, rows_b, ones_v,
             sem_g0, sem_g1, sem_s0, sem_s1, sem_c, agg_s, cnt_s) = rest
        else:
            (agg_out,
             gidx_v, sidx_v, rows_a, rows_b, ones_v,
             sem_g0, sem_g1, sem_s0, sem_s1, sem_c, agg_s) = rest
            cnt_s = None
        c = lax.axis_index("c")
        s = lax.axis_index("s")
        pltpu.sync_copy(zer, agg_s.at[pl.ds(s * _RPT, _RPT)])
        if with_counts:
            pltpu.sync_copy(zer1.at[pl.ds(s * _RPT, _RPT)],
                            cnt_s.at[pl.ds(s * _RPT, _RPT)])
        pltpu.sync_copy(ones1, ones_v)
        plsc.subcore_barrier()

        rows = (rows_a, rows_b)
        sem_g = (sem_g0, sem_g1)
        sem_s = (sem_s0, sem_s1)

        def super_group(g, carry):
            base = s * _K + g * _SG
            pltpu.sync_copy(gidx.at[pl.ds(base, _SG)], gidx_v)
            pltpu.sync_copy(sidx.at[c, pl.ds(base, _SG)], sidx_v)

            gath = [None, None]
            scat = [None, None]
            cnts = []
            gath[0] = pltpu.async_copy(yp.at[gidx_v.at[0]], rows[0], sem_g[0])
            for j in range(1, _SG + 1):
                b = j & 1
                if j < _SG:
                    if scat[b] is not None:
                        scat[b].wait()
                    gath[b] = pltpu.async_copy(
                        yp.at[gidx_v.at[j]], rows[b], sem_g[b])
                p = 1 - b
                gath[p].wait()
                scat[p] = pltpu.async_copy(
                    rows[p], agg_s.at[sidx_v.at[j - 1]], sem_s[p], add=True)
                if with_counts:
                    cnts.append(pltpu.async_copy(
                        ones_v, cnt_s.at[sidx_v.at[j - 1]], sem_c, add=True))
            scat[0].wait()
            scat[1].wait()
            for d in cnts:
                d.wait()
            return carry

        lax.fori_loop(0, _NSG, super_group, 0)
        plsc.subcore_barrier()
        pltpu.sync_copy(agg_s.at[pl.ds(s * _RPT, _RPT)],
                        agg_out.at[c, pl.ds(s * _RPT, _RPT)])
        if with_counts:
            pltpu.sync_copy(cnt_s.at[pl.ds(s * _RPT, _RPT)],
                            cnt_out.at[c, pl.ds(s * _RPT, _RPT)])

    out_type = [jax.ShapeDtypeStruct((2, _NLOC, _D), _f32)]
    scratch = [
        pltpu.VMEM((_SG, _CH), jnp.int32),
        pltpu.VMEM((_SG, _CH), jnp.int32),
        pltpu.VMEM((_CH, _D), _f32),
        pltpu.VMEM((_CH, _D), _f32),
        pltpu.VMEM((_CH,), _f32),
        pltpu.SemaphoreType.DMA,
        pltpu.SemaphoreType.DMA,
        pltpu.SemaphoreType.DMA,
        pltpu.SemaphoreType.DMA,
        pltpu.SemaphoreType.DMA,
        pltpu.VMEM_SHARED((_NLOC, _D), _f32),
    ]
    if with_counts:
        out_type = out_type + [jax.ShapeDtypeStruct((2, _NLOC), _f32)]
        scratch = scratch + [pltpu.VMEM_SHARED((_NLOC,), _f32)]
    return functools.partial(
        pl.kernel, out_type=out_type, mesh=_mesh(), scratch_types=scratch
    )(body)


def _edge_pass(yp, gidx, sidx, zer, zer1, ones1):
    return _make_edge_pass(True)(yp, gidx, sidx, zer, zer1, ones1)


def _edge_pass_nocnt(yp, gidx, sidx, zer, zer1, ones1):
    (agg,) = _make_edge_pass(False)(yp, gidx, sidx, zer, zer1, ones1)
    return agg


def _gather_body(h, idx, out, idx_v, rows_v):
    wid = lax.axis_index("s") * 2 + lax.axis_index("c")
    base = wid * (_SEL // 32)
    pltpu.sync_copy(idx.at[pl.ds(base, _SEL // 32)], idx_v)
    pltpu.sync_copy(h.at[idx_v], rows_v)
    pltpu.sync_copy(rows_v, out.at[pl.ds(base, _SEL // 32)])


def _sel_gather(h, idx):
    k = functools.partial(
        pl.kernel,
        out_type=jax.ShapeDtypeStruct((_SEL, _D), _f32),
        mesh=_mesh(),
        scratch_types=[
            pltpu.VMEM((_SEL // 32,), jnp.int32),
            pltpu.VMEM((_SEL // 32, _D), _f32),
        ],
    )(_gather_body)
    return k(h, idx)


def _mm_body(h_ref, w0_ref, w1_ref, root_ref, b_ref, yp_ref, base_ref):
    hb = h_ref[...]
    yp_ref[0] = jnp.dot(hb, w0_ref[...], preferred_element_type=_f32)
    yp_ref[1] = jnp.dot(hb, w1_ref[...], preferred_element_type=_f32)
    base_ref[...] = jnp.dot(hb, root_ref[...], preferred_element_type=_f32) + b_ref[...]


_BN = 1000


def _mm(h, w0, w1, root, b):
    wspec = pl.BlockSpec((_D, _D), lambda i: (0, 0))
    return pl.pallas_call(
        _mm_body,
        grid=(_N // _BN,),
        in_specs=[
            pl.BlockSpec((_BN, _D), lambda i: (i, 0)),
            wspec, wspec, wspec,
            pl.BlockSpec((1, _D), lambda i: (0, 0)),
        ],
        out_specs=[
            pl.BlockSpec((2, _BN, _D), lambda i: (0, i, 0)),
            pl.BlockSpec((_BN, _D), lambda i: (i, 0)),
        ],
        out_shape=[
            jax.ShapeDtypeStruct((2, _N, _D), _f32),
            jax.ShapeDtypeStruct((_N, _D), _f32),
        ],
    )(h, w0, w1, root, b.reshape(1, _D))


def _combine_body(base_ref, sums_ref, cnt_ref, h_ref):
    sm = sums_ref[0]
    cn = cnt_ref[0]
    c0 = jnp.maximum(cn[:, 0:1], 1.0)
    c1 = jnp.maximum(cn[:, 1:2], 1.0)
    o = base_ref[...] + sm[:, :_D] / c0 + sm[:, _D:] / c1
    h_ref[...] = jnp.where(o >= 0.0, o, 0.01 * o)


_CB = 1000


def _combine(base, sums, cnt):
    return pl.pallas_call(
        _combine_body,
        grid=(2, _N2 // _CB),
        in_specs=[
            pl.BlockSpec((_CB, _D), lambda c, i: (c * (_N2 // _CB) + i, 0)),
            pl.BlockSpec((1, _CB, 2 * _D), lambda c, i: (c, i, 0)),
            pl.BlockSpec((1, _CB, 2), lambda c, i: (c, i, 0)),
        ],
        out_specs=pl.BlockSpec((_CB, _D), lambda c, i: (c * (_N2 // _CB) + i, 0)),
        out_shape=jax.ShapeDtypeStruct((_N, _D), _f32),
    )(base, sums, cnt)


def _head_body(hs_ref, wm_ref, bm_ref, o_ref):
    z = jnp.dot(hs_ref[...], wm_ref[...], preferred_element_type=_f32) + bm_ref[...]
    o_ref[...] = 1.0 / (1.0 + jnp.exp(-z))


def _head(hs, wm_pad, bm_pad):
    return pl.pallas_call(
        _head_body,
        grid=(1,),
        in_specs=[
            pl.BlockSpec((_SEL, _D), lambda i: (0, 0)),
            pl.BlockSpec((_D, _D), lambda i: (0, 0)),
            pl.BlockSpec((1, _D), lambda i: (0, 0)),
        ],
        out_specs=pl.BlockSpec((_SEL, _D), lambda i: (0, 0)),
        out_shape=jax.ShapeDtypeStruct((_SEL, _D), _f32),
    )(hs, wm_pad, bm_pad)


def kernel(x, edge_index, edge_type, idx, W0, root0, b0, W1, root1, b1, Wm, bm):
    src = edge_index[0]
    dst = edge_index[1]
    et = edge_type

    gidx = et * _N + src
    half = (dst >= _N2).astype(jnp.int32)
    lidx = (dst - half * _N2) * _R + et
    trash = _TRASH + (jnp.arange(_E, dtype=jnp.int32) % (_NLOC - _TRASH))
    sidx0 = jnp.where(half == 0, lidx, trash)
    sidx1 = jnp.where(half == 1, lidx, trash)

    pad = _EPAD - _E
    padtrash = _TRASH + (jnp.arange(pad, dtype=jnp.int32) % (_NLOC - _TRASH))
    gidx = jnp.concatenate([gidx, jnp.zeros((pad,), jnp.int32)]).reshape(16 * _K, _CH)
    sidx = jnp.stack([
        jnp.concatenate([sidx0, padtrash]),
        jnp.concatenate([sidx1, padtrash]),
    ]).reshape(2, 16 * _K, _CH)

    zer = jnp.zeros((_RPT, _D), _f32)
    zer1 = jnp.zeros((_NLOC,), _f32)
    ones1 = jnp.ones((_CH,), _f32)

    wm_pad = jnp.pad(Wm, ((0, 0), (0, _D - _OUT)))
    bm_pad = jnp.pad(bm, (0, _D - _OUT)).reshape(1, _D)

    h = x
    cnts = None
    for li, (w, root, b) in enumerate(((W0, root0, b0), (W1, root1, b1))):
        yp, base = _mm(h, w[0], w[1], root, b)
        if li == 0:
            agg, cnt = _edge_pass(yp.reshape(2 * _N, _D), gidx, sidx, zer, zer1, ones1)
            cnts = cnt.reshape(2, _NLOC // 2, 2)
        else:
            agg = _edge_pass_nocnt(yp.reshape(2 * _N, _D), gidx, sidx, zer, zer1, ones1)
        sums = agg.reshape(2, _NLOC // 2, 2 * _D)
        h = _combine(base, sums, cnts)

    h_sel = _sel_gather(h, idx)
    out = _head(h_sel, wm_pad, bm_pad)[:, :_OUT]
    return (h_sel, out)
